# Initial kernel scaffold; baseline (speedup 1.0000x reference)
import jax
import jax.numpy as jnp
from jax.experimental import pallas as pl


def kernel(x, edge_index, edge_attr, graph_ids, proj_W, proj_b, en_W1, en_b1, en_W2, en_b2, conv_b, gru_W_ih, gru_b_ih, gru_W_hh, gru_b_hh, out_W, out_b):
    raise NotImplementedError("write your pallas kernel here")


# speedup vs baseline: 2.1879x; 2.1879x over previous
import functools

import jax
import jax.numpy as jnp
from jax import lax
from jax.experimental import pallas as pl
from jax.experimental.pallas import tpu as pltpu
from jax.experimental.pallas import tpu_sc as plsc

_N = 10000
_E = 160000
_G = 128
_DN = 128
_DE = 16
_H = 32
_STEPS = 2

_NW = 32
_CH = 128
_NCHUNK = 40
_PER_W = _CH * _NCHUNK
_EPAD = _NW * _PER_W
_NPW = _N // 16

_INTERPRET = False


def _linear_relu_body(x_ref, w_ref, b_ref, o_ref):
    o_ref[...] = jax.nn.relu(
        jnp.dot(x_ref[...], w_ref[...], preferred_element_type=jnp.float32)
        + b_ref[...])


def _linear_relu(x, w, b, block_rows):
    rows, din = x.shape
    dout = w.shape[1]
    grid = rows // block_rows
    return pl.pallas_call(
        _linear_relu_body,
        grid=(grid,),
        in_specs=[
            pl.BlockSpec((block_rows, din), lambda i: (i, 0)),
            pl.BlockSpec((din, dout), lambda i: (0, 0)),
            pl.BlockSpec((1, dout), lambda i: (0, 0)),
        ],
        out_specs=pl.BlockSpec((block_rows, dout), lambda i: (i, 0)),
        out_shape=jax.ShapeDtypeStruct((rows, dout), jnp.float32),
        compiler_params=pltpu.CompilerParams(
            dimension_semantics=("parallel",)),
        interpret=_INTERPRET,
    )(x, w, b)


_MSG_EB = 2048


def _msg_body(s_ref, t_ref, w2_ref, b2_ref, r_ref, o_ref):
    wg = jnp.dot(t_ref[...], w2_ref[...],
                 preferred_element_type=jnp.float32) + b2_ref[...]
    s = s_ref[...]
    srep = jnp.concatenate([s] * _H, axis=1)
    p = wg * srep
    msg = jnp.dot(p, r_ref[...], preferred_element_type=jnp.float32)
    i = pl.program_id(0)
    row = i * _MSG_EB + lax.broadcasted_iota(jnp.int32, (_MSG_EB, _H), 0)
    o_ref[...] = jnp.where(row < _E, msg, 0.0)


def _tc_msg(s_g, t, w2gt, b2g, rmat):
    grid = _EPAD // _MSG_EB
    return pl.pallas_call(
        _msg_body,
        grid=(grid,),
        in_specs=[
            pl.BlockSpec((_MSG_EB, _H), lambda i: (i, 0)),
            pl.BlockSpec((_MSG_EB, _H), lambda i: (i, 0)),
            pl.BlockSpec((_H, _H * _H), lambda i: (0, 0)),
            pl.BlockSpec((1, _H * _H), lambda i: (0, 0)),
            pl.BlockSpec((_H * _H, _H), lambda i: (0, 0)),
        ],
        out_specs=pl.BlockSpec((_MSG_EB, _H), lambda i: (i, 0)),
        out_shape=jax.ShapeDtypeStruct((_EPAD, _H), jnp.float32),
        compiler_params=pltpu.CompilerParams(
            dimension_semantics=("parallel",)),
        interpret=_INTERPRET,
    )(s_g, t, w2gt, b2g, rmat)


_GRU_RB = 1000


def _gru_body(a_ref, h_ref, cb_ref, wr_ref, wz_ref, wn_ref,
              ur_ref, uz_ref, un_ref, bi_ref, bh_ref, o_ref):
    m = jax.nn.relu(a_ref[0] + a_ref[1] + cb_ref[...])
    h = h_ref[...]

    def mm(v, w):
        return jnp.dot(v, w[...], preferred_element_type=jnp.float32)

    r = jax.nn.sigmoid(mm(m, wr_ref) + bi_ref[0:1, :]
                       + mm(h, ur_ref) + bh_ref[0:1, :])
    z = jax.nn.sigmoid(mm(m, wz_ref) + bi_ref[1:2, :]
                       + mm(h, uz_ref) + bh_ref[1:2, :])
    n = jnp.tanh(mm(m, wn_ref) + bi_ref[2:3, :]
                 + r * (mm(h, un_ref) + bh_ref[2:3, :]))
    o_ref[...] = (1.0 - z) * n + z * h


def _tc_gru(aggp, hidden, conv_b2, wr, wz, wn, ur, uz, un, bi3, bh3):
    grid = _N // _GRU_RB
    wspec = pl.BlockSpec((_H, _H), lambda i: (0, 0))
    bspec = pl.BlockSpec((3, _H), lambda i: (0, 0))
    return pl.pallas_call(
        _gru_body,
        grid=(grid,),
        in_specs=[
            pl.BlockSpec((2, _GRU_RB, _H), lambda i: (0, i, 0)),
            pl.BlockSpec((_GRU_RB, _H), lambda i: (i, 0)),
            pl.BlockSpec((1, _H), lambda i: (0, 0)),
            wspec, wspec, wspec, wspec, wspec, wspec, bspec, bspec,
        ],
        out_specs=pl.BlockSpec((_GRU_RB, _H), lambda i: (i, 0)),
        out_shape=jax.ShapeDtypeStruct((_N, _H), jnp.float32),
        compiler_params=pltpu.CompilerParams(
            dimension_semantics=("parallel",)),
        interpret=_INTERPRET,
    )(aggp, hidden, conv_b2, wr, wz, wn, ur, uz, un, bi3, bh3)


def _readout_body(n_ref, idr_ref, idc_ref, w_ref, b_ref, o_ref):
    nodes = n_ref[...]
    ids_r = idr_ref[...]
    ids_c = idc_ref[...]
    g_iota = lax.broadcasted_iota(jnp.int32, (_G, _N), 0)
    oh = (jnp.broadcast_to(ids_r, (_G, _N)) == g_iota).astype(jnp.float32)
    sum_pool = jnp.dot(oh, nodes, preferred_element_type=jnp.float32)

    neg = jnp.float32(-jnp.inf)

    def body(g, acc):
        vals = jnp.where(ids_c == g, nodes, neg)
        mg = jnp.max(vals, axis=0, keepdims=True)
        sel = lax.broadcasted_iota(jnp.int32, (_G, _H), 0) == g
        return jnp.where(sel, jnp.broadcast_to(mg, (_G, _H)), acc)

    max_pool = lax.fori_loop(0, _G, body,
                             jnp.full((_G, _H), neg, jnp.float32))
    gcat = jnp.concatenate([sum_pool, max_pool], axis=1)
    o_ref[...] = jnp.dot(gcat, w_ref[...],
                         preferred_element_type=jnp.float32) + b_ref[...]


def _tc_readout(node, ids_row, ids_col, owt, ob):
    return pl.pallas_call(
        _readout_body,
        in_specs=[
            pl.BlockSpec((_N, _H), lambda: (0, 0)),
            pl.BlockSpec((1, _N), lambda: (0, 0)),
            pl.BlockSpec((_N, 1), lambda: (0, 0)),
            pl.BlockSpec((2 * _H, 1), lambda: (0, 0)),
            pl.BlockSpec((1, 1), lambda: (0, 0)),
        ],
        out_specs=pl.BlockSpec((_G, 1), lambda: (0, 0)),
        out_shape=jax.ShapeDtypeStruct((_G, 1), jnp.float32),
        interpret=_INTERPRET,
    )(node, ids_row, ids_col, owt, ob)


def _sc_gather(table, idx_pad):
    mesh = plsc.VectorSubcoreMesh(core_axis_name="c", subcore_axis_name="s")

    @functools.partial(
        pl.kernel,
        mesh=mesh,
        out_type=jax.ShapeDtypeStruct((_EPAD, _H), jnp.float32),
        scratch_types=[
            pltpu.VMEM((_CH,), jnp.int32),
            pltpu.VMEM((_CH, _H), jnp.float32),
            pltpu.SemaphoreType.DMA,
        ],
        compiler_params=pltpu.CompilerParams(use_tc_tiling_on_sc=False),
    )
    def k(table_hbm, idx_hbm, out_hbm, idx_v, rows_v, sem):
        wid = lax.axis_index("c") * 16 + lax.axis_index("s")
        base = wid * _PER_W

        def body(j, carry):
            off = base + j * _CH
            pltpu.sync_copy(idx_hbm.at[pl.ds(off, _CH)], idx_v)
            pltpu.async_copy(table_hbm.at[idx_v], rows_v, sem).wait()
            pltpu.sync_copy(rows_v, out_hbm.at[pl.ds(off, _CH)])
            return carry

        lax.fori_loop(0, _NCHUNK, body, 0)

    return k(table, idx_pad)


def _sc_scatter(msg_pad, dst_pad, zeros_tbl):
    mesh = plsc.VectorSubcoreMesh(core_axis_name="c", subcore_axis_name="s")

    @functools.partial(
        pl.kernel,
        mesh=mesh,
        out_type=jax.ShapeDtypeStruct((2 * _N, _H), jnp.float32),
        scratch_types=[
            pltpu.VMEM((_CH,), jnp.int32),
            pltpu.VMEM((_CH, _H), jnp.float32),
            pltpu.VMEM_SHARED((_N, _H), jnp.float32),
        ],
        compiler_params=pltpu.CompilerParams(use_tc_tiling_on_sc=False),
    )
    def k(msg_hbm, dst_hbm, z_hbm, out_hbm, idx_v, rows_v, agg_sh):
        c = lax.axis_index("c")
        s = lax.axis_index("s")
        pltpu.sync_copy(z_hbm.at[pl.ds(s * _NPW, _NPW)],
                        agg_sh.at[pl.ds(s * _NPW, _NPW)])
        plsc.subcore_barrier()
        base = c * (_EPAD // 2) + s * _PER_W

        def body(j, carry):
            off = base + j * _CH
            pltpu.sync_copy(dst_hbm.at[pl.ds(off, _CH)], idx_v)
            pltpu.sync_copy(msg_hbm.at[pl.ds(off, _CH)], rows_v)
            pltpu.sync_copy(rows_v, agg_sh.at[idx_v], add=True)
            return carry

        lax.fori_loop(0, _NCHUNK, body, 0)
        plsc.subcore_barrier()
        pltpu.sync_copy(agg_sh.at[pl.ds(s * _NPW, _NPW)],
                        out_hbm.at[pl.ds(c * _N + s * _NPW, _NPW)])

    return k(msg_pad, dst_pad, zeros_tbl)


def kernel(x, edge_index, edge_attr, graph_ids, proj_W, proj_b,
           en_W1, en_b1, en_W2, en_b2, conv_b, gru_W_ih, gru_b_ih,
           gru_W_hh, gru_b_hh, out_W, out_b):
    pad = _EPAD - _E
    src_p = jnp.concatenate([edge_index[0], jnp.zeros((pad,), jnp.int32)])
    dst_p = jnp.concatenate([edge_index[1], jnp.zeros((pad,), jnp.int32)])
    ea_p = jnp.concatenate(
        [edge_attr, jnp.zeros((pad, _DE), jnp.float32)], axis=0)

    w2gt = en_W2.reshape(_H, _H, _H).transpose(2, 1, 0).reshape(_H, _H * _H)
    b2g = en_b2.reshape(_H, _H).T.reshape(1, _H * _H)
    rmat = jnp.repeat(jnp.eye(_H, dtype=jnp.float32), _H, axis=0)
    wr, wz, wn = (gru_W_ih[0:_H].T, gru_W_ih[_H:2 * _H].T,
                  gru_W_ih[2 * _H:].T)
    ur, uz, un = (gru_W_hh[0:_H].T, gru_W_hh[_H:2 * _H].T,
                  gru_W_hh[2 * _H:].T)
    bi3 = gru_b_ih.reshape(3, _H)
    bh3 = gru_b_hh.reshape(3, _H)
    zeros_tbl = jnp.zeros((_N, _H), jnp.float32)

    h = _linear_relu(x, proj_W.T, proj_b.reshape(1, _H), 1000)
    t = _linear_relu(ea_p, en_W1.T, en_b1.reshape(1, _H), 10240)

    hidden = h
    node = h
    for _ in range(_STEPS):
        s_g = _sc_gather(node, src_p)
        msg = _tc_msg(s_g, t, w2gt, b2g, rmat)
        aggp = _sc_scatter(msg, dst_p, zeros_tbl)
        hidden = _tc_gru(aggp.reshape(2, _N, _H), hidden,
                         conv_b.reshape(1, _H), wr, wz, wn, ur, uz, un,
                         bi3, bh3)
        node = hidden

    return _tc_readout(node, graph_ids.reshape(1, _N),
                       graph_ids.reshape(_N, 1), out_W.T,
                       out_b.reshape(1, 1))


# speedup vs baseline: 2.4291x; 1.1102x over previous
import functools

import jax
import jax.numpy as jnp
from jax import lax
from jax.experimental import pallas as pl
from jax.experimental.pallas import tpu as pltpu
from jax.experimental.pallas import tpu_sc as plsc

_N = 10000
_E = 160000
_G = 128
_DN = 128
_DE = 16
_H = 32
_STEPS = 2

_NW = 32
_CH = 128
_NCHUNK = 40
_PER_W = _CH * _NCHUNK
_EPAD = _NW * _PER_W
_NPW = _N // 16

_INTERPRET = False


def _linear_relu_body(x_ref, w_ref, b_ref, o_ref):
    o_ref[...] = jax.nn.relu(
        jnp.dot(x_ref[...], w_ref[...], preferred_element_type=jnp.float32)
        + b_ref[...])


def _linear_relu(x, w, b, block_rows):
    rows, din = x.shape
    dout = w.shape[1]
    grid = rows // block_rows
    return pl.pallas_call(
        _linear_relu_body,
        grid=(grid,),
        in_specs=[
            pl.BlockSpec((block_rows, din), lambda i: (i, 0)),
            pl.BlockSpec((din, dout), lambda i: (0, 0)),
            pl.BlockSpec((1, dout), lambda i: (0, 0)),
        ],
        out_specs=pl.BlockSpec((block_rows, dout), lambda i: (i, 0)),
        out_shape=jax.ShapeDtypeStruct((rows, dout), jnp.float32),
        compiler_params=pltpu.CompilerParams(
            dimension_semantics=("parallel",)),
        interpret=_INTERPRET,
    )(x, w, b)


_MSG_EB = 2048


def _msg_body(s_ref, t_ref, w2_ref, b2_ref, r_ref, o_ref):
    wg = jnp.dot(t_ref[...], w2_ref[...],
                 preferred_element_type=jnp.float32) + b2_ref[...]
    s = s_ref[...]
    srep = jnp.concatenate([s] * _H, axis=1)
    p = wg * srep
    msg = jnp.dot(p, r_ref[...], preferred_element_type=jnp.float32)
    i = pl.program_id(0)
    row = i * _MSG_EB + lax.broadcasted_iota(jnp.int32, (_MSG_EB, _H), 0)
    o_ref[...] = jnp.where(row < _E, msg, 0.0)


def _tc_msg(s_g, t, w2gt, b2g, rmat):
    grid = _EPAD // _MSG_EB
    return pl.pallas_call(
        _msg_body,
        grid=(grid,),
        in_specs=[
            pl.BlockSpec((_MSG_EB, _H), lambda i: (i, 0)),
            pl.BlockSpec((_MSG_EB, _H), lambda i: (i, 0)),
            pl.BlockSpec((_H, _H * _H), lambda i: (0, 0)),
            pl.BlockSpec((1, _H * _H), lambda i: (0, 0)),
            pl.BlockSpec((_H * _H, _H), lambda i: (0, 0)),
        ],
        out_specs=pl.BlockSpec((_MSG_EB, _H), lambda i: (i, 0)),
        out_shape=jax.ShapeDtypeStruct((_EPAD, _H), jnp.float32),
        compiler_params=pltpu.CompilerParams(
            dimension_semantics=("parallel",)),
        interpret=_INTERPRET,
    )(s_g, t, w2gt, b2g, rmat)


_GRU_RB = 1000


def _gru_body(a_ref, h_ref, cb_ref, wr_ref, wz_ref, wn_ref,
              ur_ref, uz_ref, un_ref, bi_ref, bh_ref, o_ref):
    m = jax.nn.relu(a_ref[0] + a_ref[1] + cb_ref[...])
    h = h_ref[...]

    def mm(v, w):
        return jnp.dot(v, w[...], preferred_element_type=jnp.float32)

    r = jax.nn.sigmoid(mm(m, wr_ref) + bi_ref[0:1, :]
                       + mm(h, ur_ref) + bh_ref[0:1, :])
    z = jax.nn.sigmoid(mm(m, wz_ref) + bi_ref[1:2, :]
                       + mm(h, uz_ref) + bh_ref[1:2, :])
    n = jnp.tanh(mm(m, wn_ref) + bi_ref[2:3, :]
                 + r * (mm(h, un_ref) + bh_ref[2:3, :]))
    o_ref[...] = (1.0 - z) * n + z * h


def _tc_gru(aggp, hidden, conv_b2, wr, wz, wn, ur, uz, un, bi3, bh3):
    grid = _N // _GRU_RB
    wspec = pl.BlockSpec((_H, _H), lambda i: (0, 0))
    bspec = pl.BlockSpec((3, _H), lambda i: (0, 0))
    return pl.pallas_call(
        _gru_body,
        grid=(grid,),
        in_specs=[
            pl.BlockSpec((2, _GRU_RB, _H), lambda i: (0, i, 0)),
            pl.BlockSpec((_GRU_RB, _H), lambda i: (i, 0)),
            pl.BlockSpec((1, _H), lambda i: (0, 0)),
            wspec, wspec, wspec, wspec, wspec, wspec, bspec, bspec,
        ],
        out_specs=pl.BlockSpec((_GRU_RB, _H), lambda i: (i, 0)),
        out_shape=jax.ShapeDtypeStruct((_N, _H), jnp.float32),
        compiler_params=pltpu.CompilerParams(
            dimension_semantics=("parallel",)),
        interpret=_INTERPRET,
    )(aggp, hidden, conv_b2, wr, wz, wn, ur, uz, un, bi3, bh3)


def _readout_body(n_ref, idr_ref, idc_ref, w_ref, b_ref, o_ref):
    nodes = n_ref[...]
    ids_r = idr_ref[...]
    ids_c = idc_ref[...]
    g_iota = lax.broadcasted_iota(jnp.int32, (_G, _N), 0)
    oh = (jnp.broadcast_to(ids_r, (_G, _N)) == g_iota).astype(jnp.float32)
    sum_pool = jnp.dot(oh, nodes, preferred_element_type=jnp.float32)

    neg = jnp.float32(-jnp.inf)

    def body(g, acc):
        vals = jnp.where(ids_c == g, nodes, neg)
        mg = jnp.max(vals, axis=0, keepdims=True)
        sel = lax.broadcasted_iota(jnp.int32, (_G, _H), 0) == g
        return jnp.where(sel, jnp.broadcast_to(mg, (_G, _H)), acc)

    max_pool = lax.fori_loop(0, _G, body,
                             jnp.full((_G, _H), neg, jnp.float32))
    gcat = jnp.concatenate([sum_pool, max_pool], axis=1)
    o_ref[...] = jnp.dot(gcat, w_ref[...],
                         preferred_element_type=jnp.float32) + b_ref[...]


def _tc_readout(node, ids_row, ids_col, owt, ob):
    return pl.pallas_call(
        _readout_body,
        in_specs=[
            pl.BlockSpec((_N, _H), lambda: (0, 0)),
            pl.BlockSpec((1, _N), lambda: (0, 0)),
            pl.BlockSpec((_N, 1), lambda: (0, 0)),
            pl.BlockSpec((2 * _H, 1), lambda: (0, 0)),
            pl.BlockSpec((1, 1), lambda: (0, 0)),
        ],
        out_specs=pl.BlockSpec((_G, 1), lambda: (0, 0)),
        out_shape=jax.ShapeDtypeStruct((_G, 1), jnp.float32),
        interpret=_INTERPRET,
    )(node, ids_row, ids_col, owt, ob)


_HALF = _NCHUNK // 2
_HROWS = _HALF * _CH


def _sc_gather(table, idx2d):
    mesh = plsc.VectorSubcoreMesh(core_axis_name="c", subcore_axis_name="s")

    @functools.partial(
        pl.kernel,
        mesh=mesh,
        out_type=jax.ShapeDtypeStruct((_EPAD, _H), jnp.float32),
        scratch_types=[
            pltpu.VMEM((_NCHUNK, _CH), jnp.int32),
            pltpu.VMEM((_HROWS, _H), jnp.float32),
            pltpu.SemaphoreType.DMA,
        ],
        compiler_params=pltpu.CompilerParams(use_tc_tiling_on_sc=False),
    )
    def k(table_hbm, idx_hbm, out_hbm, idx_v, big_v, sem):
        wid = lax.axis_index("c") * 16 + lax.axis_index("s")
        base = wid * _PER_W
        pltpu.sync_copy(idx_hbm.at[pl.ds(wid * _NCHUNK, _NCHUNK)], idx_v)
        for half in range(2):
            descs = [
                pltpu.async_copy(
                    table_hbm.at[idx_v.at[half * _HALF + j]],
                    big_v.at[pl.ds(j * _CH, _CH)], sem)
                for j in range(_HALF)
            ]
            for d in descs:
                d.wait()
            pltpu.sync_copy(
                big_v, out_hbm.at[pl.ds(base + half * _HROWS, _HROWS)])

    return k(table, idx2d)


def _sc_scatter(msg_pad, dst2d, zeros_tbl):
    mesh = plsc.VectorSubcoreMesh(core_axis_name="c", subcore_axis_name="s")

    @functools.partial(
        pl.kernel,
        mesh=mesh,
        out_type=jax.ShapeDtypeStruct((2 * _N, _H), jnp.float32),
        scratch_types=[
            pltpu.VMEM((_NCHUNK, _CH), jnp.int32),
            pltpu.VMEM((_HROWS, _H), jnp.float32),
            pltpu.VMEM_SHARED((_N, _H), jnp.float32),
            pltpu.SemaphoreType.DMA,
        ],
        compiler_params=pltpu.CompilerParams(use_tc_tiling_on_sc=False),
    )
    def k(msg_hbm, dst_hbm, z_hbm, out_hbm, idx_v, big_v, agg_sh, sem):
        c = lax.axis_index("c")
        s = lax.axis_index("s")
        wid = c * 16 + s
        pltpu.sync_copy(z_hbm.at[pl.ds(s * _NPW, _NPW)],
                        agg_sh.at[pl.ds(s * _NPW, _NPW)])
        pltpu.sync_copy(dst_hbm.at[pl.ds(wid * _NCHUNK, _NCHUNK)], idx_v)
        plsc.subcore_barrier()
        base = c * (_EPAD // 2) + s * _PER_W
        for half in range(2):
            pltpu.sync_copy(
                msg_hbm.at[pl.ds(base + half * _HROWS, _HROWS)], big_v)
            descs = [
                pltpu.async_copy(
                    big_v.at[pl.ds(j * _CH, _CH)],
                    agg_sh.at[idx_v.at[half * _HALF + j]],
                    sem, add=True)
                for j in range(_HALF)
            ]
            for d in descs:
                d.wait()
        plsc.subcore_barrier()
        pltpu.sync_copy(agg_sh.at[pl.ds(s * _NPW, _NPW)],
                        out_hbm.at[pl.ds(c * _N + s * _NPW, _NPW)])

    return k(msg_pad, dst2d, zeros_tbl)


def kernel(x, edge_index, edge_attr, graph_ids, proj_W, proj_b,
           en_W1, en_b1, en_W2, en_b2, conv_b, gru_W_ih, gru_b_ih,
           gru_W_hh, gru_b_hh, out_W, out_b):
    pad = _EPAD - _E
    src_p = jnp.concatenate([edge_index[0], jnp.zeros((pad,), jnp.int32)])
    dst_p = jnp.concatenate([edge_index[1], jnp.zeros((pad,), jnp.int32)])
    ea_p = jnp.concatenate(
        [edge_attr, jnp.zeros((pad, _DE), jnp.float32)], axis=0)

    w2gt = en_W2.reshape(_H, _H, _H).transpose(2, 1, 0).reshape(_H, _H * _H)
    b2g = en_b2.reshape(_H, _H).T.reshape(1, _H * _H)
    rmat = jnp.repeat(jnp.eye(_H, dtype=jnp.float32), _H, axis=0)
    wr, wz, wn = (gru_W_ih[0:_H].T, gru_W_ih[_H:2 * _H].T,
                  gru_W_ih[2 * _H:].T)
    ur, uz, un = (gru_W_hh[0:_H].T, gru_W_hh[_H:2 * _H].T,
                  gru_W_hh[2 * _H:].T)
    bi3 = gru_b_ih.reshape(3, _H)
    bh3 = gru_b_hh.reshape(3, _H)
    zeros_tbl = jnp.zeros((_N, _H), jnp.float32)

    h = _linear_relu(x, proj_W.T, proj_b.reshape(1, _H), 1000)
    t = _linear_relu(ea_p, en_W1.T, en_b1.reshape(1, _H), 10240)

    hidden = h
    node = h
    src2d = src_p.reshape(_EPAD // _CH, _CH)
    dst2d = dst_p.reshape(_EPAD // _CH, _CH)
    for _ in range(_STEPS):
        s_g = _sc_gather(node, src2d)
        msg = _tc_msg(s_g, t, w2gt, b2g, rmat)
        aggp = _sc_scatter(msg, dst2d, zeros_tbl)
        hidden = _tc_gru(aggp.reshape(2, _N, _H), hidden,
                         conv_b.reshape(1, _H), wr, wz, wn, ur, uz, un,
                         bi3, bh3)
        node = hidden

    return _tc_readout(node, graph_ids.reshape(1, _N),
                       graph_ids.reshape(_N, 1), out_W.T,
                       out_b.reshape(1, 1))


# speedup vs baseline: 2.7592x; 1.1359x over previous
import functools

import jax
import jax.numpy as jnp
from jax import lax
from jax.experimental import pallas as pl
from jax.experimental.pallas import tpu as pltpu
from jax.experimental.pallas import tpu_sc as plsc

_N = 10000
_E = 160000
_G = 128
_DN = 128
_DE = 16
_H = 32
_STEPS = 2

_NW = 32
_CH = 128
_NCHUNK = 40
_PER_W = _CH * _NCHUNK
_EPAD = _NW * _PER_W
_NPW = _N // 16

_INTERPRET = False


def _linear_relu_body(x_ref, w_ref, b_ref, o_ref):
    o_ref[...] = jax.nn.relu(
        jnp.dot(x_ref[...], w_ref[...], preferred_element_type=jnp.float32)
        + b_ref[...])


def _linear_relu(x, w, b, block_rows):
    rows, din = x.shape
    dout = w.shape[1]
    grid = rows // block_rows
    return pl.pallas_call(
        _linear_relu_body,
        grid=(grid,),
        in_specs=[
            pl.BlockSpec((block_rows, din), lambda i: (i, 0)),
            pl.BlockSpec((din, dout), lambda i: (0, 0)),
            pl.BlockSpec((1, dout), lambda i: (0, 0)),
        ],
        out_specs=pl.BlockSpec((block_rows, dout), lambda i: (i, 0)),
        out_shape=jax.ShapeDtypeStruct((rows, dout), jnp.float32),
        compiler_params=pltpu.CompilerParams(
            dimension_semantics=("parallel",)),
        interpret=_INTERPRET,
    )(x, w, b)


_MSG_EB = 1280
_MSG_REAL = _E // _MSG_EB - 1


def _msg_body(s_ref, ea_ref, w1_ref, b1_ref, w2_ref, b2_ref, r_ref, o_ref):
    t = jax.nn.relu(
        jnp.dot(ea_ref[...], w1_ref[...],
                preferred_element_type=jnp.float32) + b1_ref[...])
    tb = t.astype(jnp.bfloat16)
    wgb = (jnp.dot(tb, w2_ref[...], preferred_element_type=jnp.float32)
           .astype(jnp.bfloat16) + b2_ref[...])
    s = s_ref[...].astype(jnp.bfloat16)
    srep = jnp.concatenate([s] * _H, axis=1)
    p = wgb * srep
    msg = jnp.dot(p, r_ref[...], preferred_element_type=jnp.float32)
    i = pl.program_id(0)
    row = i * _MSG_EB + lax.broadcasted_iota(jnp.int32, (_MSG_EB, _H), 0)
    o_ref[...] = jnp.where(row < _E, msg, 0.0)


def _tc_msg(s_g, edge_attr, w1t, b1, w2gt_bf, b2g_bf, rmat_bf):
    grid = _EPAD // _MSG_EB
    return pl.pallas_call(
        _msg_body,
        grid=(grid,),
        in_specs=[
            pl.BlockSpec((_MSG_EB, _H), lambda i: (i, 0)),
            pl.BlockSpec((_MSG_EB, _DE),
                         lambda i: (jnp.minimum(i, _MSG_REAL), 0)),
            pl.BlockSpec((_DE, _H), lambda i: (0, 0)),
            pl.BlockSpec((1, _H), lambda i: (0, 0)),
            pl.BlockSpec((_H, _H * _H), lambda i: (0, 0)),
            pl.BlockSpec((1, _H * _H), lambda i: (0, 0)),
            pl.BlockSpec((_H * _H, _H), lambda i: (0, 0)),
        ],
        out_specs=pl.BlockSpec((_MSG_EB, _H), lambda i: (i, 0)),
        out_shape=jax.ShapeDtypeStruct((_EPAD, _H), jnp.float32),
        compiler_params=pltpu.CompilerParams(
            dimension_semantics=("arbitrary",)),
        interpret=_INTERPRET,
    )(s_g, edge_attr, w1t, b1, w2gt_bf, b2g_bf, rmat_bf)


_GRU_RB = 1000


def _gru_body(a0_ref, a1_ref, h_ref, cb_ref, wr_ref, wz_ref, wn_ref,
              ur_ref, uz_ref, un_ref, bi_ref, bh_ref, o_ref):
    m = jax.nn.relu(a0_ref[...] + a1_ref[...] + cb_ref[...])
    h = h_ref[...]

    def mm(v, w):
        return jnp.dot(v, w[...], preferred_element_type=jnp.float32)

    r = jax.nn.sigmoid(mm(m, wr_ref) + bi_ref[0:1, :]
                       + mm(h, ur_ref) + bh_ref[0:1, :])
    z = jax.nn.sigmoid(mm(m, wz_ref) + bi_ref[1:2, :]
                       + mm(h, uz_ref) + bh_ref[1:2, :])
    n = jnp.tanh(mm(m, wn_ref) + bi_ref[2:3, :]
                 + r * (mm(h, un_ref) + bh_ref[2:3, :]))
    o_ref[...] = (1.0 - z) * n + z * h


def _tc_gru(agg0, agg1, hidden, conv_b2, wr, wz, wn, ur, uz, un, bi3, bh3):
    grid = _N // _GRU_RB
    rspec = pl.BlockSpec((_GRU_RB, _H), lambda i: (i, 0))
    wspec = pl.BlockSpec((_H, _H), lambda i: (0, 0))
    bspec = pl.BlockSpec((3, _H), lambda i: (0, 0))
    return pl.pallas_call(
        _gru_body,
        grid=(grid,),
        in_specs=[
            rspec, rspec, rspec,
            pl.BlockSpec((1, _H), lambda i: (0, 0)),
            wspec, wspec, wspec, wspec, wspec, wspec, bspec, bspec,
        ],
        out_specs=rspec,
        out_shape=jax.ShapeDtypeStruct((_N, _H), jnp.float32),
        compiler_params=pltpu.CompilerParams(
            dimension_semantics=("parallel",)),
        interpret=_INTERPRET,
    )(agg0, agg1, hidden, conv_b2, wr, wz, wn, ur, uz, un, bi3, bh3)


def _readout_body(n_ref, nt_ref, idr_ref, w_ref, b_ref, o_ref):
    nodes = n_ref[...]
    ids_r = idr_ref[...]
    g_iota = lax.broadcasted_iota(jnp.int32, (_G, _N), 0)
    ohb = jnp.broadcast_to(ids_r, (_G, _N)) == g_iota
    sum_pool = jnp.dot(ohb.astype(jnp.float32), nodes,
                       preferred_element_type=jnp.float32)

    neg = jnp.float32(-jnp.inf)
    cols = []
    for f in range(_H):
        col = nt_ref[f:f + 1, :]
        vals = jnp.where(ohb, jnp.broadcast_to(col, (_G, _N)), neg)
        cols.append(jnp.max(vals, axis=1, keepdims=True))
    max_pool = jnp.concatenate(cols, axis=1)
    gcat = jnp.concatenate([sum_pool, max_pool], axis=1)
    o_ref[...] = jnp.dot(gcat, w_ref[...],
                         preferred_element_type=jnp.float32) + b_ref[...]


def _tc_readout(node, node_t, ids_row, owt, ob):
    return pl.pallas_call(
        _readout_body,
        in_specs=[
            pl.BlockSpec((_N, _H), lambda: (0, 0)),
            pl.BlockSpec((_H, _N), lambda: (0, 0)),
            pl.BlockSpec((1, _N), lambda: (0, 0)),
            pl.BlockSpec((2 * _H, 1), lambda: (0, 0)),
            pl.BlockSpec((1, 1), lambda: (0, 0)),
        ],
        out_specs=pl.BlockSpec((_G, 1), lambda: (0, 0)),
        out_shape=jax.ShapeDtypeStruct((_G, 1), jnp.float32),
        interpret=_INTERPRET,
    )(node, node_t, ids_row, owt, ob)


_HALF = _NCHUNK // 2
_HROWS = _HALF * _CH


def _sc_gather(table, src_p):
    mesh = plsc.VectorSubcoreMesh(core_axis_name="c", subcore_axis_name="s")

    @functools.partial(
        pl.kernel,
        mesh=mesh,
        out_type=jax.ShapeDtypeStruct((_EPAD, _H), jnp.float32),
        scratch_types=[
            pltpu.VMEM((_PER_W,), jnp.int32),
            pltpu.VMEM((_HROWS, _H), jnp.float32),
            pltpu.SemaphoreType.DMA,
        ],
        compiler_params=pltpu.CompilerParams(use_tc_tiling_on_sc=False),
    )
    def k(table_hbm, idx_hbm, out_hbm, idx_v, big_v, sem):
        wid = lax.axis_index("c") * 16 + lax.axis_index("s")
        base = wid * _PER_W
        pltpu.sync_copy(idx_hbm.at[pl.ds(base, _PER_W)], idx_v)
        for half in range(2):
            descs = [
                pltpu.async_copy(
                    table_hbm.at[idx_v.at[pl.ds((half * _HALF + j) * _CH,
                                                _CH)]],
                    big_v.at[pl.ds(j * _CH, _CH)], sem)
                for j in range(_HALF)
            ]
            for d in descs:
                d.wait()
            pltpu.sync_copy(
                big_v, out_hbm.at[pl.ds(base + half * _HROWS, _HROWS)])

    return k(table, src_p)


def _sc_scatter(msg_pad, dst_p, zeros_tbl):
    mesh = plsc.VectorSubcoreMesh(core_axis_name="c", subcore_axis_name="s")

    @functools.partial(
        pl.kernel,
        mesh=mesh,
        out_type=(jax.ShapeDtypeStruct((_N, _H), jnp.float32),
                  jax.ShapeDtypeStruct((_N, _H), jnp.float32)),
        scratch_types=[
            pltpu.VMEM((_NCHUNK, _CH), jnp.int32),
            pltpu.VMEM((_HROWS, _H), jnp.float32),
            pltpu.VMEM_SHARED((_N, _H), jnp.float32),
            pltpu.SemaphoreType.DMA,
            pltpu.SemaphoreType.DMA,
        ],
        compiler_params=pltpu.CompilerParams(use_tc_tiling_on_sc=False),
    )
    def k(msg_hbm, dst_hbm, z_hbm, out0_hbm, out1_hbm,
          idx2d, big_v, agg_sh, sem, sem2):
        c = lax.axis_index("c")
        s = lax.axis_index("s")
        wid = c * 16 + s
        base = wid * _PER_W
        pltpu.sync_copy(z_hbm.at[pl.ds(s * _NPW, _NPW)],
                        agg_sh.at[pl.ds(s * _NPW, _NPW)])
        idescs = [
            pltpu.async_copy(dst_hbm.at[pl.ds(base + j * _CH, _CH)],
                             idx2d.at[j], sem2)
            for j in range(_NCHUNK)
        ]
        for d in idescs:
            d.wait()
        plsc.subcore_barrier()
        for half in range(2):
            pltpu.sync_copy(
                msg_hbm.at[pl.ds(base + half * _HROWS, _HROWS)], big_v)
            descs = [
                pltpu.async_copy(
                    big_v.at[pl.ds(j * _CH, _CH)],
                    agg_sh.at[idx2d.at[half * _HALF + j]],
                    sem, add=True)
                for j in range(_HALF)
            ]
            for d in descs:
                d.wait()
        plsc.subcore_barrier()

        @pl.when(c == 0)
        def _():
            pltpu.sync_copy(agg_sh.at[pl.ds(s * _NPW, _NPW)],
                            out0_hbm.at[pl.ds(s * _NPW, _NPW)])

        @pl.when(c == 1)
        def _():
            pltpu.sync_copy(agg_sh.at[pl.ds(s * _NPW, _NPW)],
                            out1_hbm.at[pl.ds(s * _NPW, _NPW)])

    return k(msg_pad, dst_p, zeros_tbl)


def kernel(x, edge_index, edge_attr, graph_ids, proj_W, proj_b,
           en_W1, en_b1, en_W2, en_b2, conv_b, gru_W_ih, gru_b_ih,
           gru_W_hh, gru_b_hh, out_W, out_b):
    pad = _EPAD - _E
    src_p = jnp.concatenate([edge_index[0], jnp.zeros((pad,), jnp.int32)])
    dst_p = jnp.concatenate([edge_index[1], jnp.zeros((pad,), jnp.int32)])

    w2gt_bf = (en_W2.reshape(_H, _H, _H).transpose(2, 1, 0)
               .reshape(_H, _H * _H).astype(jnp.bfloat16))
    b2g_bf = en_b2.reshape(_H, _H).T.reshape(1, _H * _H).astype(jnp.bfloat16)
    rmat_bf = jnp.repeat(jnp.eye(_H, dtype=jnp.bfloat16), _H, axis=0)
    w1t = en_W1.T
    b1r = en_b1.reshape(1, _H)
    wr, wz, wn = (gru_W_ih[0:_H].T, gru_W_ih[_H:2 * _H].T,
                  gru_W_ih[2 * _H:].T)
    ur, uz, un = (gru_W_hh[0:_H].T, gru_W_hh[_H:2 * _H].T,
                  gru_W_hh[2 * _H:].T)
    bi3 = gru_b_ih.reshape(3, _H)
    bh3 = gru_b_hh.reshape(3, _H)
    zeros_tbl = jnp.zeros((_N, _H), jnp.float32)

    h = _linear_relu(x, proj_W.T, proj_b.reshape(1, _H), 1000)

    hidden = h
    node = h
    for _ in range(_STEPS):
        s_g = _sc_gather(node, src_p)
        msg = _tc_msg(s_g, edge_attr, w1t, b1r, w2gt_bf, b2g_bf, rmat_bf)
        agg0, agg1 = _sc_scatter(msg, dst_p, zeros_tbl)
        hidden = _tc_gru(agg0, agg1, hidden, conv_b.reshape(1, _H),
                         wr, wz, wn, ur, uz, un, bi3, bh3)
        node = hidden

    return _tc_readout(node, node.T, graph_ids.reshape(1, _N), out_W.T,
                       out_b.reshape(1, 1))
